# baseline (speedup 1.0000x reference)
import functools

import jax
import jax.numpy as jnp
from jax import lax
from jax.experimental import pallas as pl
from jax.experimental.pallas import tpu as pltpu

ACT = jnp.bfloat16
EPS = 1e-5
BN_TM = 512


def _ru(x, m):
    return (x + m - 1) // m * m


def _pick_m_tile(M):
    m8 = _ru(M, 8)
    if m8 <= 256:
        return m8, m8
    return 256, _ru(M, 256)


def _pick_n_tile(N):
    dp = _ru(N, 128)
    for t in (256, 128):
        if dp % t == 0:
            return t, dp
    return 128, dp


def _pick_k_tile(K):
    Kp = _ru(K, 128)
    if Kp <= 640:
        return Kp, Kp
    for t in (512, 384, 256):
        if Kp % t == 0:
            return t, Kp
    return 128, Kp


def _gmm_kernel(a_ref, b_ref, o_ref, acc_ref):
    @pl.when(pl.program_id(3) == 0)
    def _():
        acc_ref[...] = jnp.zeros_like(acc_ref)

    acc_ref[...] += jnp.dot(a_ref[0], b_ref[0],
                            preferred_element_type=jnp.float32)

    @pl.when(pl.program_id(3) == pl.num_programs(3) - 1)
    def _():
        o_ref[0] = acc_ref[...].astype(o_ref.dtype)


def _gmm_stats_kernel(a_ref, b_ref, o_ref, st_ref, acc_ref):
    @pl.when(pl.program_id(3) == 0)
    def _():
        acc_ref[...] = jnp.zeros_like(acc_ref)

    acc_ref[...] += jnp.dot(a_ref[0], b_ref[0],
                            preferred_element_type=jnp.float32)

    @pl.when(pl.program_id(3) == pl.num_programs(3) - 1)
    def _():
        acc = acc_ref[...]
        o_ref[0] = acc.astype(o_ref.dtype)
        colsum = jnp.sum(acc, axis=0, keepdims=True)
        colsq = jnp.sum(acc * acc, axis=0, keepdims=True)
        rows = lax.broadcasted_iota(jnp.int32, st_ref.shape[1:], 0)
        st_ref[0] = jnp.where(rows == 0, colsum,
                              jnp.where(rows == 1, colsq, 0.0))


def gmm(a, b, *, out_dtype=ACT, with_stats=False, shared_a=False):
    Ga, M, K = a.shape
    G, Kb, N = b.shape
    assert K == Kb
    a = a.astype(jnp.bfloat16)
    b = b.astype(jnp.bfloat16)
    tm, Mp = _pick_m_tile(M)
    tn, Np = _pick_n_tile(N)
    tk, Kp = _pick_k_tile(K)
    if Mp != M or Kp != K:
        a = jnp.pad(a, ((0, 0), (0, Mp - M), (0, Kp - K)))
    if Kp != K or Np != N:
        b = jnp.pad(b, ((0, 0), (0, Kp - K), (0, Np - N)))
    grid = (G, Mp // tm, Np // tn, Kp // tk)
    div = G // Ga
    if shared_a:
        a_map = lambda g, i, j, k: (g // div, i, k)
    else:
        a_map = lambda g, i, j, k: (g, i, k)
    cparams = pltpu.CompilerParams(
        dimension_semantics=("parallel", "parallel", "parallel", "arbitrary"))
    in_specs = [pl.BlockSpec((1, tm, tk), a_map),
                pl.BlockSpec((1, tk, tn), lambda g, i, j, k: (g, k, j))]

    if not with_stats:
        out = pl.pallas_call(
            _gmm_kernel,
            out_shape=jax.ShapeDtypeStruct((G, Mp, Np), out_dtype),
            grid_spec=pltpu.PrefetchScalarGridSpec(
                num_scalar_prefetch=0, grid=grid,
                in_specs=in_specs,
                out_specs=pl.BlockSpec((1, tm, tn), lambda g, i, j, k: (g, i, j)),
                scratch_shapes=[pltpu.VMEM((tm, tn), jnp.float32)]),
            compiler_params=cparams,
        )(a, b)
        return out[:, :M, :N]

    mt = Mp // tm
    out, st = pl.pallas_call(
        _gmm_stats_kernel,
        out_shape=(jax.ShapeDtypeStruct((G, Mp, Np), out_dtype),
                   jax.ShapeDtypeStruct((G, mt * 8, Np), jnp.float32)),
        grid_spec=pltpu.PrefetchScalarGridSpec(
            num_scalar_prefetch=0, grid=grid,
            in_specs=in_specs,
            out_specs=(pl.BlockSpec((1, tm, tn), lambda g, i, j, k: (g, i, j)),
                       pl.BlockSpec((1, 8, tn), lambda g, i, j, k: (g, i, j))),
            scratch_shapes=[pltpu.VMEM((tm, tn), jnp.float32)]),
        compiler_params=cparams,
    )(a, b)
    st = st.reshape(G, mt, 8, Np)
    col_sum = jnp.sum(st[:, :, 0, :N], axis=1)
    col_sq = jnp.sum(st[:, :, 1, :N], axis=1)
    mean = col_sum / M
    var = jnp.maximum(col_sq / M - mean * mean, 0.0)
    return out[:, :M, :N], mean, var


def _make_bn_kernel(relu, has_res):
    if has_res:
        def _bn(x_ref, s_ref, b_ref, r_ref, o_ref):
            y = x_ref[0].astype(jnp.float32) * s_ref[0] + b_ref[0]
            y = y + r_ref[0].astype(jnp.float32)
            if relu:
                y = jnp.maximum(y, 0.0)
            o_ref[0] = y.astype(o_ref.dtype)
    else:
        def _bn(x_ref, s_ref, b_ref, o_ref):
            y = x_ref[0].astype(jnp.float32) * s_ref[0] + b_ref[0]
            if relu:
                y = jnp.maximum(y, 0.0)
            o_ref[0] = y.astype(o_ref.dtype)
    return _bn


def gbn(x, mean, var, relu, residual=None):
    G, M, C = x.shape
    scale = (1.0 / jnp.sqrt(var + EPS)).astype(jnp.float32)
    bias = (-mean * scale).astype(jnp.float32)

    fold = 128 // C if (C < 128 and 128 % C == 0) else 1
    Mf = _ru(M, fold)
    Cf = C * fold

    def prep(t):
        if Mf != M:
            t = jnp.pad(t, ((0, 0), (0, Mf - M), (0, 0)))
        return t.reshape(G, Mf // fold, Cf)

    xf = prep(x)
    rf = prep(residual) if residual is not None else None
    rows = Mf // fold
    tm = min(BN_TM, _ru(rows, 8))
    rows_p = _ru(rows, tm)
    if rows_p != rows:
        xf = jnp.pad(xf, ((0, 0), (0, rows_p - rows), (0, 0)))
        if rf is not None:
            rf = jnp.pad(rf, ((0, 0), (0, rows_p - rows), (0, 0)))
    s = jnp.tile(scale.reshape(G, 1, C), (1, 1, fold))
    b = jnp.tile(bias.reshape(G, 1, C), (1, 1, fold))

    in_specs = [pl.BlockSpec((1, tm, Cf), lambda g, i: (g, i, 0)),
                pl.BlockSpec((1, 1, Cf), lambda g, i: (g, 0, 0)),
                pl.BlockSpec((1, 1, Cf), lambda g, i: (g, 0, 0))]
    args = [xf, s, b]
    if rf is not None:
        in_specs.append(pl.BlockSpec((1, tm, Cf), lambda g, i: (g, i, 0)))
        args.append(rf)

    y = pl.pallas_call(
        _make_bn_kernel(relu, rf is not None),
        out_shape=jax.ShapeDtypeStruct((G, rows_p, Cf), ACT),
        grid=(G, rows_p // tm),
        in_specs=in_specs,
        out_specs=pl.BlockSpec((1, tm, Cf), lambda g, i: (g, i, 0)),
        compiler_params=pltpu.CompilerParams(
            dimension_semantics=("parallel", "parallel")),
    )(*args)
    return y[:, :rows].reshape(G, rows * fold, C)[:, :M]


def _make_iconv_kernel(H, W, C):
    def _iconv(x_ref, s_ref, b_ref, w_ref, o_ref, st_ref):
        Bc = x_ref.shape[1]
        xb = x_ref[0].astype(jnp.float32)
        ih = lax.broadcasted_iota(jnp.int32, xb.shape, 1)
        iw = lax.broadcasted_iota(jnp.int32, xb.shape, 2)
        valid = (ih >= 1) & (ih <= H) & (iw >= 1) & (iw <= W)
        xn = jnp.where(valid,
                       jnp.maximum(xb * s_ref[0, 0] + b_ref[0, 0], 0.0),
                       0.0).astype(jnp.bfloat16)
        acc = None
        for t in range(9):
            dy, dx = t // 3, t % 3
            xs = xn[:, dy:dy + H, dx:dx + W, :].reshape(Bc * H * W, C)
            wv = w_ref[0, t * C:(t + 1) * C, :]
            d = jnp.dot(xs, wv, preferred_element_type=jnp.float32)
            acc = d if acc is None else acc + d
        o_ref[0] = acc[:, :o_ref.shape[2]].astype(o_ref.dtype)
        colsum = jnp.sum(acc, axis=0, keepdims=True)
        colsq = jnp.sum(acc * acc, axis=0, keepdims=True)
        rows = lax.broadcasted_iota(jnp.int32, st_ref.shape[1:], 0)
        st_ref[0] = jnp.where(rows == 0, colsum,
                              jnp.where(rows == 1, colsq, 0.0))
    return _iconv


def iconv_g(x, wmat, mean=None, var=None):
    G, B, H, W, C = x.shape
    _, K, N = wmat.shape
    Np = _ru(N, 128)
    w = wmat.astype(jnp.bfloat16)
    if Np != N:
        w = jnp.pad(w, ((0, 0), (0, 0), (0, Np - N)))
    if mean is None:
        scale = jnp.ones((G, 1, C), jnp.float32)
        bias = jnp.zeros((G, 1, C), jnp.float32)
    else:
        s = 1.0 / jnp.sqrt(var + EPS)
        scale = s.reshape(G, 1, C).astype(jnp.float32)
        bias = (-mean * s).reshape(G, 1, C).astype(jnp.float32)
    NC = 1
    while B % (NC * 2) == 0 and (B // NC) * H * W * Np * 4 > 8 * 1024 * 1024:
        NC *= 2
    Bc = B // NC
    xp = jnp.pad(x.astype(jnp.bfloat16),
                 ((0, 0), (0, 0), (1, 1), (1, 1), (0, 0)))
    xp = xp.reshape(G * NC, Bc, H + 2, W + 2, C)
    Mc = Bc * H * W
    out, st = pl.pallas_call(
        _make_iconv_kernel(H, W, C),
        out_shape=(jax.ShapeDtypeStruct((G * NC, Mc, Np), ACT),
                   jax.ShapeDtypeStruct((G * NC, 8, Np), jnp.float32)),
        grid=(G * NC,),
        in_specs=[pl.BlockSpec((1, Bc, H + 2, W + 2, C),
                               lambda i: (i, 0, 0, 0, 0)),
                  pl.BlockSpec((1, 1, C), lambda i: (i // NC, 0, 0)),
                  pl.BlockSpec((1, 1, C), lambda i: (i // NC, 0, 0)),
                  pl.BlockSpec((1, K, Np), lambda i: (i // NC, 0, 0))],
        out_specs=(pl.BlockSpec((1, Mc, Np), lambda i: (i, 0, 0)),
                   pl.BlockSpec((1, 8, Np), lambda i: (i, 0, 0))),
        compiler_params=pltpu.CompilerParams(
            dimension_semantics=("parallel",)),
    )(xp, scale, bias, w)
    M = B * H * W
    out2d = out[:, :, :N].reshape(G, M, N)
    st = st.reshape(G, NC, 8, Np).sum(axis=1)
    mean_o = st[:, 0, :N] / M
    var_o = jnp.maximum(st[:, 1, :N] / M - mean_o * mean_o, 0.0)
    return out2d, mean_o, var_o, (B, H, W, N)


def _make_block_tail_kernel(H, W, C, M):
    HW = H * W

    def _tail(x_ref, s1_ref, b1_ref, w_ref, id_ref, sd_ref, bd_ref, o_ref,
              r2_ref):
        B = x_ref.shape[1]
        s1 = s1_ref[0, 0]
        b1 = b1_ref[0, 0]
        colsum = jnp.zeros((1, C), jnp.float32)
        colsq = jnp.zeros((1, C), jnp.float32)
        for b in range(B):
            xb = x_ref[0, b].astype(jnp.float32)
            ih = lax.broadcasted_iota(jnp.int32, xb.shape, 0)
            iw = lax.broadcasted_iota(jnp.int32, xb.shape, 1)
            valid = (ih >= 1) & (ih <= H) & (iw >= 1) & (iw <= W)
            xn = jnp.where(valid, jnp.maximum(xb * s1 + b1, 0.0),
                           0.0).astype(jnp.bfloat16)
            acc = None
            for t in range(9):
                dy, dx = t // 3, t % 3
                xs = xn[dy:dy + H, dx:dx + W, :].reshape(HW, C)
                wv = w_ref[0, t * C:(t + 1) * C, :]
                d = jnp.dot(xs, wv, preferred_element_type=jnp.float32)
                acc = d if acc is None else acc + d
            acc = acc[:, :C]
            colsum += jnp.sum(acc, axis=0, keepdims=True)
            colsq += jnp.sum(acc * acc, axis=0, keepdims=True)
            r2_ref[b * HW:(b + 1) * HW, :] = acc.astype(jnp.bfloat16)
        mean = colsum / M
        var = jnp.maximum(colsq / M - mean * mean, 0.0)
        s2 = lax.rsqrt(var + EPS)
        b2 = -mean * s2
        for b in range(B):
            rows = slice(b * HW, (b + 1) * HW)
            idv = (id_ref[0, rows, :].astype(jnp.float32) * sd_ref[0, 0]
                   + bd_ref[0, 0])
            y = jnp.maximum(r2_ref[rows, :].astype(jnp.float32) * s2 + b2
                            + idv, 0.0)
            o_ref[0, rows, :] = y.astype(o_ref.dtype)
    return _tail


def block_tail_g(raw1_sp, m1, v1, w2, id2d, id_mean=None, id_var=None):
    G, B, H, W, C = raw1_sp.shape
    _, K, N = w2.shape
    assert N == C
    Np = _ru(N, 128)
    M = B * H * W
    w = w2.astype(jnp.bfloat16)
    if Np != N:
        w = jnp.pad(w, ((0, 0), (0, 0), (0, Np - N)))
    s1 = 1.0 / jnp.sqrt(v1 + EPS)
    scale1 = s1.reshape(G, 1, C).astype(jnp.float32)
    bias1 = (-m1 * s1).reshape(G, 1, C).astype(jnp.float32)
    if id_mean is None:
        sd = jnp.ones((G, 1, C), jnp.float32)
        bd = jnp.zeros((G, 1, C), jnp.float32)
    else:
        sdv = 1.0 / jnp.sqrt(id_var + EPS)
        sd = sdv.reshape(G, 1, C).astype(jnp.float32)
        bd = (-id_mean * sdv).reshape(G, 1, C).astype(jnp.float32)
    xp = jnp.pad(raw1_sp.astype(jnp.bfloat16),
                 ((0, 0), (0, 0), (1, 1), (1, 1), (0, 0)))
    y = pl.pallas_call(
        _make_block_tail_kernel(H, W, C, M),
        out_shape=jax.ShapeDtypeStruct((G, M, C), ACT),
        grid=(G,),
        in_specs=[pl.BlockSpec((1, B, H + 2, W + 2, C),
                               lambda i: (i, 0, 0, 0, 0)),
                  pl.BlockSpec((1, 1, C), lambda i: (i, 0, 0)),
                  pl.BlockSpec((1, 1, C), lambda i: (i, 0, 0)),
                  pl.BlockSpec((1, K, Np), lambda i: (i, 0, 0)),
                  pl.BlockSpec((1, M, C), lambda i: (i, 0, 0)),
                  pl.BlockSpec((1, 1, C), lambda i: (i, 0, 0)),
                  pl.BlockSpec((1, 1, C), lambda i: (i, 0, 0))],
        out_specs=pl.BlockSpec((1, M, C), lambda i: (i, 0, 0)),
        scratch_shapes=[pltpu.VMEM((M, C), jnp.bfloat16)],
        compiler_params=pltpu.CompilerParams(
            dimension_semantics=("parallel",)),
    )(xp, scale1, bias1, w, id2d.astype(jnp.bfloat16), sd, bd)
    return y


def _im2col(x, k, stride, pad, pad_k_to=0):
    G, B, H, W, C = x.shape
    if pad:
        x = jnp.pad(x, ((0, 0), (0, 0), (pad, pad), (pad, pad), (0, 0)))
    Ho = (H + 2 * pad - k) // stride + 1
    Wo = (W + 2 * pad - k) // stride + 1
    cols = []
    for i in range(k):
        for j in range(k):
            cols.append(x[:, :, i:i + stride * Ho:stride,
                          j:j + stride * Wo:stride, :])
    K = k * k * C
    if pad_k_to > K:
        cols.append(jnp.zeros((G, B, Ho, Wo, pad_k_to - K), x.dtype))
        K = pad_k_to
    patches = jnp.concatenate(cols, axis=-1).reshape(G, B * Ho * Wo, K)
    return patches, (B, Ho, Wo)


def conv_g(x, wmat, stride, pad):
    G, B, H, W, C = x.shape
    _, K, Cout = wmat.shape
    k = int(round((K // C) ** 0.5))
    patches, (B_, Ho, Wo) = _im2col(x, k, stride, pad)
    out2d, mean, var = gmm(patches, wmat, with_stats=True)
    return out2d, mean, var, (B_, Ho, Wo, Cout)


def maxpool_3x3_s2(x):
    return lax.reduce_window(x, jnp.asarray(-jnp.inf, x.dtype), lax.max,
                             (1, 1, 3, 3, 1), (1, 1, 2, 2, 1),
                             ((0, 0), (0, 0), (1, 1), (1, 1), (0, 0)))


def basic_block_g(x, wd, w1, w2, stride):
    G, B, H, W, C = x.shape
    if wd is not None:
        id2d, dm, dv, _ = conv_g(x, wd, stride, 0)
    else:
        id2d, dm, dv = x.reshape(G, B * H * W, C), None, None
    if stride == 1:
        raw1, m1, v1, shp = iconv_g(x, w1)
    else:
        raw1, m1, v1, shp = conv_g(x, w1, stride, 1)
    if shp[-1] >= 128:
        y = block_tail_g(raw1.reshape(G, *shp), m1, v1, w2, id2d,
                         id_mean=dm, id_var=dv)
    else:
        if dm is not None:
            id2d = gbn(id2d, dm, dv, relu=False)
        out2d, m2, v2, shp = iconv_g(raw1.reshape(G, *shp), w2,
                                     mean=m1, var=v1)
        y = gbn(out2d, m2, v2, relu=True, residual=id2d)
    return y.reshape(G, *shp)


@jax.jit
def _forward(x1, x2, stacks, fc1_w, fc1_b, fc2_w, fc2_b, fcf_w, fcf_b):
    x1h = jnp.transpose(x1, (0, 2, 3, 1)).astype(ACT)
    x2h = jnp.transpose(x2, (0, 2, 3, 1)).astype(ACT)

    xin = jnp.stack([x1h, x2h])
    patches, (B, Ho, Wo) = _im2col(xin, 7, 2, 3, pad_k_to=256)
    w1s = stacks['conv1_w']
    w1s = jnp.pad(w1s, ((0, 0), (0, 256 - w1s.shape[1]), (0, 0)))
    out2d, mean, var = gmm(patches, w1s, with_stats=True, shared_a=True)
    Cout = stacks['conv1_w'].shape[-1]
    raw = out2d.reshape(4, B, Ho, Wo, Cout)
    pooled = maxpool_3x3_s2(raw)
    G_, B_, Hp_, Wp_, _ = pooled.shape
    x = gbn(pooled.reshape(4, B_ * Hp_ * Wp_, Cout), mean, var,
            relu=True).reshape(4, B_, Hp_, Wp_, Cout)

    for li, (nb, stride) in enumerate(zip((2, 2, 2, 3), (1, 2, 2, 2))):
        for bi in range(nb):
            key = f'layer{li + 1}_{bi}'
            x = basic_block_g(x, stacks.get(key + '_down'),
                              stacks[key + '_conv1'], stacks[key + '_conv2'],
                              stride if bi == 0 else 1)

    pooled = x.astype(jnp.float32).mean(axis=(2, 3))
    a_big = jnp.concatenate([pooled[0], pooled[2], pooled[1], pooled[3]],
                            axis=1)
    n1 = fc1_w.shape[1]
    w_big = jnp.zeros((2048, 3 * n1), jnp.bfloat16)
    w_big = w_big.at[0:512, 0:n1].set(fc1_w.astype(jnp.bfloat16))
    w_big = w_big.at[512:1024, n1:2 * n1].set(fc2_w.astype(jnp.bfloat16))
    w_big = w_big.at[1024:2048, 2 * n1:3 * n1].set(fcf_w.astype(jnp.bfloat16))
    out = gmm(a_big[None], w_big[None], out_dtype=jnp.float32)[0]
    x1_lin = out[:, 0:n1] + fc1_b
    x2_lin = out[:, n1:2 * n1] + fc2_b
    x_out = out[:, 2 * n1:3 * n1] + fcf_b
    return x1_lin, x2_lin, x_out


def kernel(x1, x2, *args):
    names = _ARG_NAMES
    p = dict(zip(names, args))

    def stack4(fmt):
        return jnp.stack([p[fmt.format(net)].astype(jnp.bfloat16)
                          for net in _NETS])

    stacks = {'conv1_w': stack4('{}__conv1_w')}
    for li, nb in enumerate((2, 2, 2, 3)):
        for bi in range(nb):
            key = f'layer{li + 1}_{bi}'
            base = '{}__layer%d__%d__' % (li + 1, bi)
            if bi == 0 and li > 0:
                stacks[key + '_down'] = stack4(base + 'down_w')
            stacks[key + '_conv1'] = stack4(base + 'conv1_w')
            stacks[key + '_conv2'] = stack4(base + 'conv2_w')

    return _forward(x1, x2, stacks,
                    p['ska1__res1__fc_w'], p['ska1__res1__fc_b'],
                    p['ska2__res1__fc_w'], p['ska2__res1__fc_b'],
                    p['fc_w'], p['fc_b'])


_NETS = ('ska1__res1', 'ska1__res2', 'ska2__res1', 'ska2__res2')


def _build_arg_names():
    names = []
    for ska in ('ska1', 'ska2'):
        for res in ('res1', 'res2'):
            pre = f'{ska}__{res}'
            names.append(f'{pre}__conv1_w')
            for li, nb in enumerate((2, 2, 2, 3)):
                for bi in range(nb):
                    if bi == 0 and li > 0:
                        names.append(f'{pre}__layer{li + 1}__{bi}__down_w')
                    names.append(f'{pre}__layer{li + 1}__{bi}__conv1_w')
                    names.append(f'{pre}__layer{li + 1}__{bi}__conv2_w')
            names.append(f'{pre}__fc_w')
            names.append(f'{pre}__fc_b')
    names.append('fc_w')
    names.append('fc_b')
    return tuple(names)


_ARG_NAMES = _build_arg_names()


# speedup vs baseline: 1.5518x; 1.5518x over previous
import functools

import jax
import jax.numpy as jnp
from jax import lax
from jax.experimental import pallas as pl
from jax.experimental.pallas import tpu as pltpu

ACT = jnp.bfloat16
EPS = 1e-5
BN_TM = 512


def _ru(x, m):
    return (x + m - 1) // m * m


def _pick_m_tile(M):
    m8 = _ru(M, 8)
    if m8 <= 256:
        return m8, m8
    return 256, _ru(M, 256)


def _pick_n_tile(N):
    dp = _ru(N, 128)
    for t in (256, 128):
        if dp % t == 0:
            return t, dp
    return 128, dp


def _pick_k_tile(K):
    Kp = _ru(K, 128)
    if Kp <= 640:
        return Kp, Kp
    for t in (512, 384, 256):
        if Kp % t == 0:
            return t, Kp
    return 128, Kp


def _gmm_kernel(a_ref, b_ref, o_ref, acc_ref):
    @pl.when(pl.program_id(3) == 0)
    def _():
        acc_ref[...] = jnp.zeros_like(acc_ref)

    acc_ref[...] += jnp.dot(a_ref[0], b_ref[0],
                            preferred_element_type=jnp.float32)

    @pl.when(pl.program_id(3) == pl.num_programs(3) - 1)
    def _():
        o_ref[0] = acc_ref[...].astype(o_ref.dtype)


def _gmm_stats_kernel(a_ref, b_ref, o_ref, st_ref, acc_ref):
    @pl.when(pl.program_id(3) == 0)
    def _():
        acc_ref[...] = jnp.zeros_like(acc_ref)

    acc_ref[...] += jnp.dot(a_ref[0], b_ref[0],
                            preferred_element_type=jnp.float32)

    @pl.when(pl.program_id(3) == pl.num_programs(3) - 1)
    def _():
        acc = acc_ref[...]
        o_ref[0] = acc.astype(o_ref.dtype)
        colsum = jnp.sum(acc, axis=0, keepdims=True)
        colsq = jnp.sum(acc * acc, axis=0, keepdims=True)
        rows = lax.broadcasted_iota(jnp.int32, st_ref.shape[1:], 0)
        st_ref[0] = jnp.where(rows == 0, colsum,
                              jnp.where(rows == 1, colsq, 0.0))


def gmm(a, b, *, out_dtype=ACT, with_stats=False, shared_a=False):
    Ga, M, K = a.shape
    G, Kb, N = b.shape
    assert K == Kb
    a = a.astype(jnp.bfloat16)
    b = b.astype(jnp.bfloat16)
    tm, Mp = _pick_m_tile(M)
    tn, Np = _pick_n_tile(N)
    tk, Kp = _pick_k_tile(K)
    if Mp != M or Kp != K:
        a = jnp.pad(a, ((0, 0), (0, Mp - M), (0, Kp - K)))
    if Kp != K or Np != N:
        b = jnp.pad(b, ((0, 0), (0, Kp - K), (0, Np - N)))
    grid = (G, Mp // tm, Np // tn, Kp // tk)
    div = G // Ga
    if shared_a:
        a_map = lambda g, i, j, k: (g // div, i, k)
    else:
        a_map = lambda g, i, j, k: (g, i, k)
    cparams = pltpu.CompilerParams(
        dimension_semantics=("parallel", "parallel", "parallel", "arbitrary"))
    in_specs = [pl.BlockSpec((1, tm, tk), a_map),
                pl.BlockSpec((1, tk, tn), lambda g, i, j, k: (g, k, j))]

    if not with_stats:
        out = pl.pallas_call(
            _gmm_kernel,
            out_shape=jax.ShapeDtypeStruct((G, Mp, Np), out_dtype),
            grid_spec=pltpu.PrefetchScalarGridSpec(
                num_scalar_prefetch=0, grid=grid,
                in_specs=in_specs,
                out_specs=pl.BlockSpec((1, tm, tn), lambda g, i, j, k: (g, i, j)),
                scratch_shapes=[pltpu.VMEM((tm, tn), jnp.float32)]),
            compiler_params=cparams,
        )(a, b)
        return out[:, :M, :N]

    mt = Mp // tm
    out, st = pl.pallas_call(
        _gmm_stats_kernel,
        out_shape=(jax.ShapeDtypeStruct((G, Mp, Np), out_dtype),
                   jax.ShapeDtypeStruct((G, mt * 8, Np), jnp.float32)),
        grid_spec=pltpu.PrefetchScalarGridSpec(
            num_scalar_prefetch=0, grid=grid,
            in_specs=in_specs,
            out_specs=(pl.BlockSpec((1, tm, tn), lambda g, i, j, k: (g, i, j)),
                       pl.BlockSpec((1, 8, tn), lambda g, i, j, k: (g, i, j))),
            scratch_shapes=[pltpu.VMEM((tm, tn), jnp.float32)]),
        compiler_params=cparams,
    )(a, b)
    st = st.reshape(G, mt, 8, Np)
    col_sum = jnp.sum(st[:, :, 0, :N], axis=1)
    col_sq = jnp.sum(st[:, :, 1, :N], axis=1)
    mean = col_sum / M
    var = jnp.maximum(col_sq / M - mean * mean, 0.0)
    return out[:, :M, :N], mean, var


def _make_bn_kernel(relu, has_res):
    if has_res:
        def _bn(x_ref, s_ref, b_ref, r_ref, o_ref):
            y = x_ref[0].astype(jnp.float32) * s_ref[0] + b_ref[0]
            y = y + r_ref[0].astype(jnp.float32)
            if relu:
                y = jnp.maximum(y, 0.0)
            o_ref[0] = y.astype(o_ref.dtype)
    else:
        def _bn(x_ref, s_ref, b_ref, o_ref):
            y = x_ref[0].astype(jnp.float32) * s_ref[0] + b_ref[0]
            if relu:
                y = jnp.maximum(y, 0.0)
            o_ref[0] = y.astype(o_ref.dtype)
    return _bn


def gbn(x, mean, var, relu, residual=None):
    G, M, C = x.shape
    scale = (1.0 / jnp.sqrt(var + EPS)).astype(jnp.float32)
    bias = (-mean * scale).astype(jnp.float32)

    fold = 128 // C if (C < 128 and 128 % C == 0) else 1
    Mf = _ru(M, fold)
    Cf = C * fold

    def prep(t):
        if Mf != M:
            t = jnp.pad(t, ((0, 0), (0, Mf - M), (0, 0)))
        return t.reshape(G, Mf // fold, Cf)

    xf = prep(x)
    rf = prep(residual) if residual is not None else None
    rows = Mf // fold
    tm = min(BN_TM, _ru(rows, 8))
    rows_p = _ru(rows, tm)
    if rows_p != rows:
        xf = jnp.pad(xf, ((0, 0), (0, rows_p - rows), (0, 0)))
        if rf is not None:
            rf = jnp.pad(rf, ((0, 0), (0, rows_p - rows), (0, 0)))
    s = jnp.tile(scale.reshape(G, 1, C), (1, 1, fold))
    b = jnp.tile(bias.reshape(G, 1, C), (1, 1, fold))

    in_specs = [pl.BlockSpec((1, tm, Cf), lambda g, i: (g, i, 0)),
                pl.BlockSpec((1, 1, Cf), lambda g, i: (g, 0, 0)),
                pl.BlockSpec((1, 1, Cf), lambda g, i: (g, 0, 0))]
    args = [xf, s, b]
    if rf is not None:
        in_specs.append(pl.BlockSpec((1, tm, Cf), lambda g, i: (g, i, 0)))
        args.append(rf)

    y = pl.pallas_call(
        _make_bn_kernel(relu, rf is not None),
        out_shape=jax.ShapeDtypeStruct((G, rows_p, Cf), ACT),
        grid=(G, rows_p // tm),
        in_specs=in_specs,
        out_specs=pl.BlockSpec((1, tm, Cf), lambda g, i: (g, i, 0)),
        compiler_params=pltpu.CompilerParams(
            dimension_semantics=("parallel", "parallel")),
    )(*args)
    return y[:, :rows].reshape(G, rows * fold, C)[:, :M]


def _make_iconv_kernel(H, W, C):
    def _iconv(x_ref, s_ref, b_ref, w_ref, o_ref, st_ref):
        Bc = x_ref.shape[1]
        xb = x_ref[0].astype(jnp.float32)
        ih = lax.broadcasted_iota(jnp.int32, xb.shape, 1)
        iw = lax.broadcasted_iota(jnp.int32, xb.shape, 2)
        valid = (ih >= 1) & (ih <= H) & (iw >= 1) & (iw <= W)
        xn = jnp.where(valid,
                       jnp.maximum(xb * s_ref[0, 0] + b_ref[0, 0], 0.0),
                       0.0).astype(jnp.bfloat16)
        acc = None
        for t in range(9):
            dy, dx = t // 3, t % 3
            xs = xn[:, dy:dy + H, dx:dx + W, :].reshape(Bc * H * W, C)
            wv = w_ref[0, t * C:(t + 1) * C, :]
            d = jnp.dot(xs, wv, preferred_element_type=jnp.float32)
            acc = d if acc is None else acc + d
        o_ref[0] = acc[:, :o_ref.shape[2]].astype(o_ref.dtype)
        colsum = jnp.sum(acc, axis=0, keepdims=True)
        colsq = jnp.sum(acc * acc, axis=0, keepdims=True)
        rows = lax.broadcasted_iota(jnp.int32, st_ref.shape[1:], 0)
        st_ref[0] = jnp.where(rows == 0, colsum,
                              jnp.where(rows == 1, colsq, 0.0))
    return _iconv


def iconv_g(x, wmat, mean=None, var=None):
    G, B, H, W, C = x.shape
    _, K, N = wmat.shape
    Np = _ru(N, 128)
    w = wmat.astype(jnp.bfloat16)
    if Np != N:
        w = jnp.pad(w, ((0, 0), (0, 0), (0, Np - N)))
    if mean is None:
        scale = jnp.ones((G, 1, C), jnp.float32)
        bias = jnp.zeros((G, 1, C), jnp.float32)
    else:
        s = 1.0 / jnp.sqrt(var + EPS)
        scale = s.reshape(G, 1, C).astype(jnp.float32)
        bias = (-mean * s).reshape(G, 1, C).astype(jnp.float32)
    NC = 1
    while B % (NC * 2) == 0 and (B // NC) * H * W * Np * 4 > 8 * 1024 * 1024:
        NC *= 2
    Bc = B // NC
    xp = jnp.pad(x.astype(jnp.bfloat16),
                 ((0, 0), (0, 0), (1, 1), (1, 1), (0, 0)))
    xp = xp.reshape(G * NC, Bc, H + 2, W + 2, C)
    Mc = Bc * H * W
    out, st = pl.pallas_call(
        _make_iconv_kernel(H, W, C),
        out_shape=(jax.ShapeDtypeStruct((G * NC, Mc, Np), ACT),
                   jax.ShapeDtypeStruct((G * NC, 8, Np), jnp.float32)),
        grid=(G * NC,),
        in_specs=[pl.BlockSpec((1, Bc, H + 2, W + 2, C),
                               lambda i: (i, 0, 0, 0, 0)),
                  pl.BlockSpec((1, 1, C), lambda i: (i // NC, 0, 0)),
                  pl.BlockSpec((1, 1, C), lambda i: (i // NC, 0, 0)),
                  pl.BlockSpec((1, K, Np), lambda i: (i // NC, 0, 0))],
        out_specs=(pl.BlockSpec((1, Mc, Np), lambda i: (i, 0, 0)),
                   pl.BlockSpec((1, 8, Np), lambda i: (i, 0, 0))),
        compiler_params=pltpu.CompilerParams(
            dimension_semantics=("parallel",)),
    )(xp, scale, bias, w)
    M = B * H * W
    out2d = out[:, :, :N].reshape(G, M, N)
    st = st.reshape(G, NC, 8, Np).sum(axis=1)
    mean_o = st[:, 0, :N] / M
    var_o = jnp.maximum(st[:, 1, :N] / M - mean_o * mean_o, 0.0)
    return out2d, mean_o, var_o, (B, H, W, N)


def _make_block_tail_kernel(H, W, C, M):
    HW = H * W

    def _tail(x_ref, s1_ref, b1_ref, w_ref, id_ref, sd_ref, bd_ref, o_ref,
              r2_ref):
        B = x_ref.shape[1]
        s1 = s1_ref[0, 0]
        b1 = b1_ref[0, 0]
        colsum = jnp.zeros((1, C), jnp.float32)
        colsq = jnp.zeros((1, C), jnp.float32)
        for b in range(B):
            xb = x_ref[0, b].astype(jnp.float32)
            ih = lax.broadcasted_iota(jnp.int32, xb.shape, 0)
            iw = lax.broadcasted_iota(jnp.int32, xb.shape, 1)
            valid = (ih >= 1) & (ih <= H) & (iw >= 1) & (iw <= W)
            xn = jnp.where(valid, jnp.maximum(xb * s1 + b1, 0.0),
                           0.0).astype(jnp.bfloat16)
            acc = None
            for t in range(9):
                dy, dx = t // 3, t % 3
                xs = xn[dy:dy + H, dx:dx + W, :].reshape(HW, C)
                wv = w_ref[0, t * C:(t + 1) * C, :]
                d = jnp.dot(xs, wv, preferred_element_type=jnp.float32)
                acc = d if acc is None else acc + d
            acc = acc[:, :C]
            colsum += jnp.sum(acc, axis=0, keepdims=True)
            colsq += jnp.sum(acc * acc, axis=0, keepdims=True)
            r2_ref[b * HW:(b + 1) * HW, :] = acc.astype(jnp.bfloat16)
        mean = colsum / M
        var = jnp.maximum(colsq / M - mean * mean, 0.0)
        s2 = lax.rsqrt(var + EPS)
        b2 = -mean * s2
        for b in range(B):
            rows = slice(b * HW, (b + 1) * HW)
            idv = (id_ref[0, rows, :].astype(jnp.float32) * sd_ref[0, 0]
                   + bd_ref[0, 0])
            y = jnp.maximum(r2_ref[rows, :].astype(jnp.float32) * s2 + b2
                            + idv, 0.0)
            o_ref[0, rows, :] = y.astype(o_ref.dtype)
    return _tail


def block_tail_g(raw1_sp, m1, v1, w2, id2d, id_mean=None, id_var=None):
    G, B, H, W, C = raw1_sp.shape
    _, K, N = w2.shape
    assert N == C
    Np = _ru(N, 128)
    M = B * H * W
    w = w2.astype(jnp.bfloat16)
    if Np != N:
        w = jnp.pad(w, ((0, 0), (0, 0), (0, Np - N)))
    s1 = 1.0 / jnp.sqrt(v1 + EPS)
    scale1 = s1.reshape(G, 1, C).astype(jnp.float32)
    bias1 = (-m1 * s1).reshape(G, 1, C).astype(jnp.float32)
    if id_mean is None:
        sd = jnp.ones((G, 1, C), jnp.float32)
        bd = jnp.zeros((G, 1, C), jnp.float32)
    else:
        sdv = 1.0 / jnp.sqrt(id_var + EPS)
        sd = sdv.reshape(G, 1, C).astype(jnp.float32)
        bd = (-id_mean * sdv).reshape(G, 1, C).astype(jnp.float32)
    xp = jnp.pad(raw1_sp.astype(jnp.bfloat16),
                 ((0, 0), (0, 0), (1, 1), (1, 1), (0, 0)))
    y = pl.pallas_call(
        _make_block_tail_kernel(H, W, C, M),
        out_shape=jax.ShapeDtypeStruct((G, M, C), ACT),
        grid=(G,),
        in_specs=[pl.BlockSpec((1, B, H + 2, W + 2, C),
                               lambda i: (i, 0, 0, 0, 0)),
                  pl.BlockSpec((1, 1, C), lambda i: (i, 0, 0)),
                  pl.BlockSpec((1, 1, C), lambda i: (i, 0, 0)),
                  pl.BlockSpec((1, K, Np), lambda i: (i, 0, 0)),
                  pl.BlockSpec((1, M, C), lambda i: (i, 0, 0)),
                  pl.BlockSpec((1, 1, C), lambda i: (i, 0, 0)),
                  pl.BlockSpec((1, 1, C), lambda i: (i, 0, 0))],
        out_specs=pl.BlockSpec((1, M, C), lambda i: (i, 0, 0)),
        scratch_shapes=[pltpu.VMEM((M, C), jnp.bfloat16)],
        compiler_params=pltpu.CompilerParams(
            dimension_semantics=("parallel",)),
    )(xp, scale1, bias1, w, id2d.astype(jnp.bfloat16), sd, bd)
    return y


def _im2col(x, k, stride, pad, pad_k_to=0):
    G, B, H, W, C = x.shape
    if pad:
        x = jnp.pad(x, ((0, 0), (0, 0), (pad, pad), (pad, pad), (0, 0)))
    Ho = (H + 2 * pad - k) // stride + 1
    Wo = (W + 2 * pad - k) // stride + 1
    cols = []
    for i in range(k):
        for j in range(k):
            cols.append(x[:, :, i:i + stride * Ho:stride,
                          j:j + stride * Wo:stride, :])
    K = k * k * C
    if pad_k_to > K:
        cols.append(jnp.zeros((G, B, Ho, Wo, pad_k_to - K), x.dtype))
        K = pad_k_to
    patches = jnp.concatenate(cols, axis=-1).reshape(G, B * Ho * Wo, K)
    return patches, (B, Ho, Wo)


def conv_g(x, wmat, stride, pad):
    G, B, H, W, C = x.shape
    _, K, Cout = wmat.shape
    k = int(round((K // C) ** 0.5))
    patches, (B_, Ho, Wo) = _im2col(x, k, stride, pad)
    out2d, mean, var = gmm(patches, wmat, with_stats=True)
    return out2d, mean, var, (B_, Ho, Wo, Cout)


def maxpool_3x3_s2(x):
    return lax.reduce_window(x, jnp.asarray(-jnp.inf, x.dtype), lax.max,
                             (1, 1, 3, 3, 1), (1, 1, 2, 2, 1),
                             ((0, 0), (0, 0), (1, 1), (1, 1), (0, 0)))


def basic_block_g(x, wd, w1, w2, stride):
    G, B, H, W, C = x.shape
    if wd is not None:
        id2d, dm, dv, _ = conv_g(x, wd, stride, 0)
    else:
        id2d, dm, dv = x.reshape(G, B * H * W, C), None, None
    if stride == 1:
        raw1, m1, v1, shp = iconv_g(x, w1)
    else:
        raw1, m1, v1, shp = conv_g(x, w1, stride, 1)
    if shp[-1] >= 128:
        y = block_tail_g(raw1.reshape(G, *shp), m1, v1, w2, id2d,
                         id_mean=dm, id_var=dv)
    else:
        if dm is not None:
            id2d = gbn(id2d, dm, dv, relu=False)
        out2d, m2, v2, shp = iconv_g(raw1.reshape(G, *shp), w2,
                                     mean=m1, var=v1)
        y = gbn(out2d, m2, v2, relu=True, residual=id2d)
    return y.reshape(G, *shp)


@jax.jit
def _forward(x1, x2, stacks, fc1_w, fc1_b, fc2_w, fc2_b, fcf_w, fcf_b):
    x1h = jnp.transpose(x1, (0, 2, 3, 1)).astype(ACT)
    x2h = jnp.transpose(x2, (0, 2, 3, 1)).astype(ACT)

    xin = jnp.stack([x1h, x2h])
    patches, (B, Ho, Wo) = _im2col(xin, 7, 2, 3)
    out2d, mean, var = gmm(patches, stacks['conv1_w'], with_stats=True,
                           shared_a=True)
    Cout = stacks['conv1_w'].shape[-1]
    raw = out2d.reshape(4, B, Ho, Wo, Cout)
    pooled = maxpool_3x3_s2(raw)
    G_, B_, Hp_, Wp_, _ = pooled.shape
    x = gbn(pooled.reshape(4, B_ * Hp_ * Wp_, Cout), mean, var,
            relu=True).reshape(4, B_, Hp_, Wp_, Cout)

    for li, (nb, stride) in enumerate(zip((2, 2, 2, 3), (1, 2, 2, 2))):
        for bi in range(nb):
            key = f'layer{li + 1}_{bi}'
            x = basic_block_g(x, stacks.get(key + '_down'),
                              stacks[key + '_conv1'], stacks[key + '_conv2'],
                              stride if bi == 0 else 1)

    pooled = x.astype(jnp.float32).mean(axis=(2, 3))
    a_big = jnp.concatenate([pooled[0], pooled[2], pooled[1], pooled[3]],
                            axis=1)
    n1 = fc1_w.shape[1]
    w_big = jnp.zeros((2048, 3 * n1), jnp.bfloat16)
    w_big = w_big.at[0:512, 0:n1].set(fc1_w.astype(jnp.bfloat16))
    w_big = w_big.at[512:1024, n1:2 * n1].set(fc2_w.astype(jnp.bfloat16))
    w_big = w_big.at[1024:2048, 2 * n1:3 * n1].set(fcf_w.astype(jnp.bfloat16))
    out = gmm(a_big[None], w_big[None], out_dtype=jnp.float32)[0]
    x1_lin = out[:, 0:n1] + fc1_b
    x2_lin = out[:, n1:2 * n1] + fc2_b
    x_out = out[:, 2 * n1:3 * n1] + fcf_b
    return x1_lin, x2_lin, x_out


def kernel(x1, x2, *args):
    names = _ARG_NAMES
    p = dict(zip(names, args))

    def stack4(fmt):
        return jnp.stack([p[fmt.format(net)].astype(jnp.bfloat16)
                          for net in _NETS])

    stacks = {'conv1_w': stack4('{}__conv1_w')}
    for li, nb in enumerate((2, 2, 2, 3)):
        for bi in range(nb):
            key = f'layer{li + 1}_{bi}'
            base = '{}__layer%d__%d__' % (li + 1, bi)
            if bi == 0 and li > 0:
                stacks[key + '_down'] = stack4(base + 'down_w')
            stacks[key + '_conv1'] = stack4(base + 'conv1_w')
            stacks[key + '_conv2'] = stack4(base + 'conv2_w')

    return _forward(x1, x2, stacks,
                    p['ska1__res1__fc_w'], p['ska1__res1__fc_b'],
                    p['ska2__res1__fc_w'], p['ska2__res1__fc_b'],
                    p['fc_w'], p['fc_b'])


_NETS = ('ska1__res1', 'ska1__res2', 'ska2__res1', 'ska2__res2')


def _build_arg_names():
    names = []
    for ska in ('ska1', 'ska2'):
        for res in ('res1', 'res2'):
            pre = f'{ska}__{res}'
            names.append(f'{pre}__conv1_w')
            for li, nb in enumerate((2, 2, 2, 3)):
                for bi in range(nb):
                    if bi == 0 and li > 0:
                        names.append(f'{pre}__layer{li + 1}__{bi}__down_w')
                    names.append(f'{pre}__layer{li + 1}__{bi}__conv1_w')
                    names.append(f'{pre}__layer{li + 1}__{bi}__conv2_w')
            names.append(f'{pre}__fc_w')
            names.append(f'{pre}__fc_b')
    names.append('fc_w')
    names.append('fc_b')
    return tuple(names)


_ARG_NAMES = _build_arg_names()


# speedup vs baseline: 1.7111x; 1.1027x over previous
import functools

import jax
import jax.numpy as jnp
from jax import lax
from jax.experimental import pallas as pl
from jax.experimental.pallas import tpu as pltpu

ACT = jnp.bfloat16
EPS = 1e-5
BN_TM = 512


def _ru(x, m):
    return (x + m - 1) // m * m


def _pick_m_tile(M):
    m8 = _ru(M, 8)
    if m8 <= 256:
        return m8, m8
    if M >= 50000 and M % 1024 == 0:
        return 1024, M
    return 256, _ru(M, 256)


def _pick_n_tile(N):
    dp = _ru(N, 128)
    for t in (256, 128):
        if dp % t == 0:
            return t, dp
    return 128, dp


def _pick_k_tile(K):
    Kp = _ru(K, 128)
    if Kp <= 640:
        return Kp, Kp
    for t in (512, 384, 256):
        if Kp % t == 0:
            return t, Kp
    return 128, Kp


def _gmm_kernel(a_ref, b_ref, o_ref, acc_ref):
    @pl.when(pl.program_id(3) == 0)
    def _():
        acc_ref[...] = jnp.zeros_like(acc_ref)

    acc_ref[...] += jnp.dot(a_ref[0], b_ref[0],
                            preferred_element_type=jnp.float32)

    @pl.when(pl.program_id(3) == pl.num_programs(3) - 1)
    def _():
        o_ref[0] = acc_ref[...].astype(o_ref.dtype)


def _gmm_stats_kernel(a_ref, b_ref, o_ref, st_ref, acc_ref):
    @pl.when(pl.program_id(3) == 0)
    def _():
        acc_ref[...] = jnp.zeros_like(acc_ref)

    acc_ref[...] += jnp.dot(a_ref[0], b_ref[0],
                            preferred_element_type=jnp.float32)

    @pl.when(pl.program_id(3) == pl.num_programs(3) - 1)
    def _():
        acc = acc_ref[...]
        o_ref[0] = acc.astype(o_ref.dtype)
        colsum = jnp.sum(acc, axis=0, keepdims=True)
        colsq = jnp.sum(acc * acc, axis=0, keepdims=True)
        rows = lax.broadcasted_iota(jnp.int32, st_ref.shape[1:], 0)
        st_ref[0] = jnp.where(rows == 0, colsum,
                              jnp.where(rows == 1, colsq, 0.0))


def gmm(a, b, *, out_dtype=ACT, with_stats=False, shared_a=False):
    Ga, M, K = a.shape
    G, Kb, N = b.shape
    assert K == Kb
    a = a.astype(jnp.bfloat16)
    b = b.astype(jnp.bfloat16)
    tm, Mp = _pick_m_tile(M)
    tn, Np = _pick_n_tile(N)
    tk, Kp = _pick_k_tile(K)
    if Mp != M or Kp != K:
        a = jnp.pad(a, ((0, 0), (0, Mp - M), (0, Kp - K)))
    if Kp != K or Np != N:
        b = jnp.pad(b, ((0, 0), (0, Kp - K), (0, Np - N)))
    grid = (G, Mp // tm, Np // tn, Kp // tk)
    div = G // Ga
    if shared_a:
        a_map = lambda g, i, j, k: (g // div, i, k)
    else:
        a_map = lambda g, i, j, k: (g, i, k)
    cparams = pltpu.CompilerParams(
        dimension_semantics=("parallel", "parallel", "parallel", "arbitrary"))
    in_specs = [pl.BlockSpec((1, tm, tk), a_map),
                pl.BlockSpec((1, tk, tn), lambda g, i, j, k: (g, k, j))]

    if not with_stats:
        out = pl.pallas_call(
            _gmm_kernel,
            out_shape=jax.ShapeDtypeStruct((G, Mp, Np), out_dtype),
            grid_spec=pltpu.PrefetchScalarGridSpec(
                num_scalar_prefetch=0, grid=grid,
                in_specs=in_specs,
                out_specs=pl.BlockSpec((1, tm, tn), lambda g, i, j, k: (g, i, j)),
                scratch_shapes=[pltpu.VMEM((tm, tn), jnp.float32)]),
            compiler_params=cparams,
        )(a, b)
        return out[:, :M, :N]

    mt = Mp // tm
    out, st = pl.pallas_call(
        _gmm_stats_kernel,
        out_shape=(jax.ShapeDtypeStruct((G, Mp, Np), out_dtype),
                   jax.ShapeDtypeStruct((G, mt * 8, Np), jnp.float32)),
        grid_spec=pltpu.PrefetchScalarGridSpec(
            num_scalar_prefetch=0, grid=grid,
            in_specs=in_specs,
            out_specs=(pl.BlockSpec((1, tm, tn), lambda g, i, j, k: (g, i, j)),
                       pl.BlockSpec((1, 8, tn), lambda g, i, j, k: (g, i, j))),
            scratch_shapes=[pltpu.VMEM((tm, tn), jnp.float32)]),
        compiler_params=cparams,
    )(a, b)
    st = st.reshape(G, mt, 8, Np)
    col_sum = jnp.sum(st[:, :, 0, :N], axis=1)
    col_sq = jnp.sum(st[:, :, 1, :N], axis=1)
    mean = col_sum / M
    var = jnp.maximum(col_sq / M - mean * mean, 0.0)
    return out[:, :M, :N], mean, var


def _make_bn_kernel(relu, has_res):
    if has_res:
        def _bn(x_ref, s_ref, b_ref, r_ref, o_ref):
            y = x_ref[0].astype(jnp.float32) * s_ref[0] + b_ref[0]
            y = y + r_ref[0].astype(jnp.float32)
            if relu:
                y = jnp.maximum(y, 0.0)
            o_ref[0] = y.astype(o_ref.dtype)
    else:
        def _bn(x_ref, s_ref, b_ref, o_ref):
            y = x_ref[0].astype(jnp.float32) * s_ref[0] + b_ref[0]
            if relu:
                y = jnp.maximum(y, 0.0)
            o_ref[0] = y.astype(o_ref.dtype)
    return _bn


def gbn(x, mean, var, relu, residual=None):
    G, M, C = x.shape
    scale = (1.0 / jnp.sqrt(var + EPS)).astype(jnp.float32)
    bias = (-mean * scale).astype(jnp.float32)

    fold = 128 // C if (C < 128 and 128 % C == 0) else 1
    Mf = _ru(M, fold)
    Cf = C * fold

    def prep(t):
        if Mf != M:
            t = jnp.pad(t, ((0, 0), (0, Mf - M), (0, 0)))
        return t.reshape(G, Mf // fold, Cf)

    xf = prep(x)
    rf = prep(residual) if residual is not None else None
    rows = Mf // fold
    tm = min(BN_TM, _ru(rows, 8))
    rows_p = _ru(rows, tm)
    if rows_p != rows:
        xf = jnp.pad(xf, ((0, 0), (0, rows_p - rows), (0, 0)))
        if rf is not None:
            rf = jnp.pad(rf, ((0, 0), (0, rows_p - rows), (0, 0)))
    s = jnp.tile(scale.reshape(G, 1, C), (1, 1, fold))
    b = jnp.tile(bias.reshape(G, 1, C), (1, 1, fold))

    in_specs = [pl.BlockSpec((1, tm, Cf), lambda g, i: (g, i, 0)),
                pl.BlockSpec((1, 1, Cf), lambda g, i: (g, 0, 0)),
                pl.BlockSpec((1, 1, Cf), lambda g, i: (g, 0, 0))]
    args = [xf, s, b]
    if rf is not None:
        in_specs.append(pl.BlockSpec((1, tm, Cf), lambda g, i: (g, i, 0)))
        args.append(rf)

    y = pl.pallas_call(
        _make_bn_kernel(relu, rf is not None),
        out_shape=jax.ShapeDtypeStruct((G, rows_p, Cf), ACT),
        grid=(G, rows_p // tm),
        in_specs=in_specs,
        out_specs=pl.BlockSpec((1, tm, Cf), lambda g, i: (g, i, 0)),
        compiler_params=pltpu.CompilerParams(
            dimension_semantics=("parallel", "parallel")),
    )(*args)
    return y[:, :rows].reshape(G, rows * fold, C)[:, :M]


def _make_iconv_kernel(H, W, C):
    def _iconv(x_ref, s_ref, b_ref, w_ref, o_ref, st_ref):
        Bc = x_ref.shape[1]
        xb = x_ref[0].astype(jnp.float32)
        ih = lax.broadcasted_iota(jnp.int32, xb.shape, 1)
        iw = lax.broadcasted_iota(jnp.int32, xb.shape, 2)
        valid = (ih >= 1) & (ih <= H) & (iw >= 1) & (iw <= W)
        xn = jnp.where(valid,
                       jnp.maximum(xb * s_ref[0, 0] + b_ref[0, 0], 0.0),
                       0.0).astype(jnp.bfloat16)
        acc = None
        for t in range(9):
            dy, dx = t // 3, t % 3
            xs = xn[:, dy:dy + H, dx:dx + W, :].reshape(Bc * H * W, C)
            wv = w_ref[0, t * C:(t + 1) * C, :]
            d = jnp.dot(xs, wv, preferred_element_type=jnp.float32)
            acc = d if acc is None else acc + d
        o_ref[0] = acc[:, :o_ref.shape[2]].astype(o_ref.dtype)
        colsum = jnp.sum(acc, axis=0, keepdims=True)
        colsq = jnp.sum(acc * acc, axis=0, keepdims=True)
        rows = lax.broadcasted_iota(jnp.int32, st_ref.shape[1:], 0)
        st_ref[0] = jnp.where(rows == 0, colsum,
                              jnp.where(rows == 1, colsq, 0.0))
    return _iconv


def iconv_g(x, wmat, mean=None, var=None):
    G, B, H, W, C = x.shape
    _, K, N = wmat.shape
    Np = _ru(N, 128)
    w = wmat.astype(jnp.bfloat16)
    if Np != N:
        w = jnp.pad(w, ((0, 0), (0, 0), (0, Np - N)))
    if mean is None:
        scale = jnp.ones((G, 1, C), jnp.float32)
        bias = jnp.zeros((G, 1, C), jnp.float32)
    else:
        s = 1.0 / jnp.sqrt(var + EPS)
        scale = s.reshape(G, 1, C).astype(jnp.float32)
        bias = (-mean * s).reshape(G, 1, C).astype(jnp.float32)
    NC = 1
    while B % (NC * 2) == 0 and (B // NC) * H * W * Np * 4 > 8 * 1024 * 1024:
        NC *= 2
    Bc = B // NC
    xp = jnp.pad(x.astype(jnp.bfloat16),
                 ((0, 0), (0, 0), (1, 1), (1, 1), (0, 0)))
    xp = xp.reshape(G * NC, Bc, H + 2, W + 2, C)
    Mc = Bc * H * W
    out, st = pl.pallas_call(
        _make_iconv_kernel(H, W, C),
        out_shape=(jax.ShapeDtypeStruct((G * NC, Mc, Np), ACT),
                   jax.ShapeDtypeStruct((G * NC, 8, Np), jnp.float32)),
        grid=(G * NC,),
        in_specs=[pl.BlockSpec((1, Bc, H + 2, W + 2, C),
                               lambda i: (i, 0, 0, 0, 0)),
                  pl.BlockSpec((1, 1, C), lambda i: (i // NC, 0, 0)),
                  pl.BlockSpec((1, 1, C), lambda i: (i // NC, 0, 0)),
                  pl.BlockSpec((1, K, Np), lambda i: (i // NC, 0, 0))],
        out_specs=(pl.BlockSpec((1, Mc, Np), lambda i: (i, 0, 0)),
                   pl.BlockSpec((1, 8, Np), lambda i: (i, 0, 0))),
        compiler_params=pltpu.CompilerParams(
            dimension_semantics=("parallel",)),
    )(xp, scale, bias, w)
    M = B * H * W
    out2d = out[:, :, :N].reshape(G, M, N)
    st = st.reshape(G, NC, 8, Np).sum(axis=1)
    mean_o = st[:, 0, :N] / M
    var_o = jnp.maximum(st[:, 1, :N] / M - mean_o * mean_o, 0.0)
    return out2d, mean_o, var_o, (B, H, W, N)


def _make_block_tail_kernel(H, W, C, M):
    HW = H * W

    def _tail(x_ref, s1_ref, b1_ref, w_ref, id_ref, sd_ref, bd_ref, o_ref,
              r2_ref):
        B = x_ref.shape[1]
        s1 = s1_ref[0, 0]
        b1 = b1_ref[0, 0]
        colsum = jnp.zeros((1, C), jnp.float32)
        colsq = jnp.zeros((1, C), jnp.float32)
        for b in range(B):
            xb = x_ref[0, b].astype(jnp.float32)
            ih = lax.broadcasted_iota(jnp.int32, xb.shape, 0)
            iw = lax.broadcasted_iota(jnp.int32, xb.shape, 1)
            valid = (ih >= 1) & (ih <= H) & (iw >= 1) & (iw <= W)
            xn = jnp.where(valid, jnp.maximum(xb * s1 + b1, 0.0),
                           0.0).astype(jnp.bfloat16)
            acc = None
            for t in range(9):
                dy, dx = t // 3, t % 3
                xs = xn[dy:dy + H, dx:dx + W, :].reshape(HW, C)
                wv = w_ref[0, t * C:(t + 1) * C, :]
                d = jnp.dot(xs, wv, preferred_element_type=jnp.float32)
                acc = d if acc is None else acc + d
            acc = acc[:, :C]
            colsum += jnp.sum(acc, axis=0, keepdims=True)
            colsq += jnp.sum(acc * acc, axis=0, keepdims=True)
            r2_ref[b * HW:(b + 1) * HW, :] = acc.astype(jnp.bfloat16)
        mean = colsum / M
        var = jnp.maximum(colsq / M - mean * mean, 0.0)
        s2 = lax.rsqrt(var + EPS)
        b2 = -mean * s2
        for b in range(B):
            rows = slice(b * HW, (b + 1) * HW)
            idv = (id_ref[0, rows, :].astype(jnp.float32) * sd_ref[0, 0]
                   + bd_ref[0, 0])
            y = jnp.maximum(r2_ref[rows, :].astype(jnp.float32) * s2 + b2
                            + idv, 0.0)
            o_ref[0, rows, :] = y.astype(o_ref.dtype)
    return _tail


def block_tail_g(raw1_sp, m1, v1, w2, id2d, id_mean=None, id_var=None):
    G, B, H, W, C = raw1_sp.shape
    _, K, N = w2.shape
    assert N == C
    Np = _ru(N, 128)
    M = B * H * W
    w = w2.astype(jnp.bfloat16)
    if Np != N:
        w = jnp.pad(w, ((0, 0), (0, 0), (0, Np - N)))
    s1 = 1.0 / jnp.sqrt(v1 + EPS)
    scale1 = s1.reshape(G, 1, C).astype(jnp.float32)
    bias1 = (-m1 * s1).reshape(G, 1, C).astype(jnp.float32)
    if id_mean is None:
        sd = jnp.ones((G, 1, C), jnp.float32)
        bd = jnp.zeros((G, 1, C), jnp.float32)
    else:
        sdv = 1.0 / jnp.sqrt(id_var + EPS)
        sd = sdv.reshape(G, 1, C).astype(jnp.float32)
        bd = (-id_mean * sdv).reshape(G, 1, C).astype(jnp.float32)
    xp = jnp.pad(raw1_sp.astype(jnp.bfloat16),
                 ((0, 0), (0, 0), (1, 1), (1, 1), (0, 0)))
    y = pl.pallas_call(
        _make_block_tail_kernel(H, W, C, M),
        out_shape=jax.ShapeDtypeStruct((G, M, C), ACT),
        grid=(G,),
        in_specs=[pl.BlockSpec((1, B, H + 2, W + 2, C),
                               lambda i: (i, 0, 0, 0, 0)),
                  pl.BlockSpec((1, 1, C), lambda i: (i, 0, 0)),
                  pl.BlockSpec((1, 1, C), lambda i: (i, 0, 0)),
                  pl.BlockSpec((1, K, Np), lambda i: (i, 0, 0)),
                  pl.BlockSpec((1, M, C), lambda i: (i, 0, 0)),
                  pl.BlockSpec((1, 1, C), lambda i: (i, 0, 0)),
                  pl.BlockSpec((1, 1, C), lambda i: (i, 0, 0))],
        out_specs=pl.BlockSpec((1, M, C), lambda i: (i, 0, 0)),
        scratch_shapes=[pltpu.VMEM((M, C), jnp.bfloat16)],
        compiler_params=pltpu.CompilerParams(
            dimension_semantics=("parallel",)),
    )(xp, scale1, bias1, w, id2d.astype(jnp.bfloat16), sd, bd)
    return y


def _im2col(x, k, stride, pad, pad_k_to=0):
    G, B, H, W, C = x.shape
    if pad:
        x = jnp.pad(x, ((0, 0), (0, 0), (pad, pad), (pad, pad), (0, 0)))
    Ho = (H + 2 * pad - k) // stride + 1
    Wo = (W + 2 * pad - k) // stride + 1
    cols = []
    for i in range(k):
        for j in range(k):
            cols.append(x[:, :, i:i + stride * Ho:stride,
                          j:j + stride * Wo:stride, :])
    K = k * k * C
    if pad_k_to > K:
        cols.append(jnp.zeros((G, B, Ho, Wo, pad_k_to - K), x.dtype))
        K = pad_k_to
    patches = jnp.concatenate(cols, axis=-1).reshape(G, B * Ho * Wo, K)
    return patches, (B, Ho, Wo)


def conv_g(x, wmat, stride, pad):
    G, B, H, W, C = x.shape
    _, K, Cout = wmat.shape
    k = int(round((K // C) ** 0.5))
    patches, (B_, Ho, Wo) = _im2col(x, k, stride, pad)
    out2d, mean, var = gmm(patches, wmat, with_stats=True)
    return out2d, mean, var, (B_, Ho, Wo, Cout)


def maxpool_3x3_s2(x):
    return lax.reduce_window(x, jnp.asarray(-jnp.inf, x.dtype), lax.max,
                             (1, 1, 3, 3, 1), (1, 1, 2, 2, 1),
                             ((0, 0), (0, 0), (1, 1), (1, 1), (0, 0)))


def basic_block_g(x, wd, w1, w2, stride):
    G, B, H, W, C = x.shape
    if wd is not None:
        id2d, dm, dv, _ = conv_g(x, wd, stride, 0)
    else:
        id2d, dm, dv = x.reshape(G, B * H * W, C), None, None
    if stride == 1:
        raw1, m1, v1, shp = iconv_g(x, w1)
    else:
        raw1, m1, v1, shp = conv_g(x, w1, stride, 1)
    if shp[-1] >= 128:
        y = block_tail_g(raw1.reshape(G, *shp), m1, v1, w2, id2d,
                         id_mean=dm, id_var=dv)
    else:
        if dm is not None:
            id2d = gbn(id2d, dm, dv, relu=False)
        out2d, m2, v2, shp = iconv_g(raw1.reshape(G, *shp), w2,
                                     mean=m1, var=v1)
        y = gbn(out2d, m2, v2, relu=True, residual=id2d)
    return y.reshape(G, *shp)


@jax.jit
def _forward(x1, x2, stacks, fc1_w, fc1_b, fc2_w, fc2_b, fcf_w, fcf_b):
    x1h = jnp.transpose(x1, (0, 2, 3, 1)).astype(ACT)
    x2h = jnp.transpose(x2, (0, 2, 3, 1)).astype(ACT)

    xin = jnp.stack([x1h, x2h])
    patches, (B, Ho, Wo) = _im2col(xin, 7, 2, 3)
    out2d, mean, var = gmm(patches, stacks['conv1_w'], with_stats=True,
                           shared_a=True)
    Cout = stacks['conv1_w'].shape[-1]
    raw = out2d.reshape(4, B, Ho, Wo, Cout)
    pooled = maxpool_3x3_s2(raw)
    G_, B_, Hp_, Wp_, _ = pooled.shape
    x = gbn(pooled.reshape(4, B_ * Hp_ * Wp_, Cout), mean, var,
            relu=True).reshape(4, B_, Hp_, Wp_, Cout)

    for li, (nb, stride) in enumerate(zip((2, 2, 2, 3), (1, 2, 2, 2))):
        for bi in range(nb):
            key = f'layer{li + 1}_{bi}'
            x = basic_block_g(x, stacks.get(key + '_down'),
                              stacks[key + '_conv1'], stacks[key + '_conv2'],
                              stride if bi == 0 else 1)

    pooled = x.astype(jnp.float32).mean(axis=(2, 3))
    a_big = jnp.concatenate([pooled[0], pooled[2], pooled[1], pooled[3]],
                            axis=1)
    n1 = fc1_w.shape[1]
    w_big = jnp.zeros((2048, 3 * n1), jnp.bfloat16)
    w_big = w_big.at[0:512, 0:n1].set(fc1_w.astype(jnp.bfloat16))
    w_big = w_big.at[512:1024, n1:2 * n1].set(fc2_w.astype(jnp.bfloat16))
    w_big = w_big.at[1024:2048, 2 * n1:3 * n1].set(fcf_w.astype(jnp.bfloat16))
    out = gmm(a_big[None], w_big[None], out_dtype=jnp.float32)[0]
    x1_lin = out[:, 0:n1] + fc1_b
    x2_lin = out[:, n1:2 * n1] + fc2_b
    x_out = out[:, 2 * n1:3 * n1] + fcf_b
    return x1_lin, x2_lin, x_out


def kernel(x1, x2, *args):
    names = _ARG_NAMES
    p = dict(zip(names, args))

    def stack4(fmt):
        return jnp.stack([p[fmt.format(net)].astype(jnp.bfloat16)
                          for net in _NETS])

    stacks = {'conv1_w': stack4('{}__conv1_w')}
    for li, nb in enumerate((2, 2, 2, 3)):
        for bi in range(nb):
            key = f'layer{li + 1}_{bi}'
            base = '{}__layer%d__%d__' % (li + 1, bi)
            if bi == 0 and li > 0:
                stacks[key + '_down'] = stack4(base + 'down_w')
            stacks[key + '_conv1'] = stack4(base + 'conv1_w')
            stacks[key + '_conv2'] = stack4(base + 'conv2_w')

    return _forward(x1, x2, stacks,
                    p['ska1__res1__fc_w'], p['ska1__res1__fc_b'],
                    p['ska2__res1__fc_w'], p['ska2__res1__fc_b'],
                    p['fc_w'], p['fc_b'])


_NETS = ('ska1__res1', 'ska1__res2', 'ska2__res1', 'ska2__res2')


def _build_arg_names():
    names = []
    for ska in ('ska1', 'ska2'):
        for res in ('res1', 'res2'):
            pre = f'{ska}__{res}'
            names.append(f'{pre}__conv1_w')
            for li, nb in enumerate((2, 2, 2, 3)):
                for bi in range(nb):
                    if bi == 0 and li > 0:
                        names.append(f'{pre}__layer{li + 1}__{bi}__down_w')
                    names.append(f'{pre}__layer{li + 1}__{bi}__conv1_w')
                    names.append(f'{pre}__layer{li + 1}__{bi}__conv2_w')
            names.append(f'{pre}__fc_w')
            names.append(f'{pre}__fc_b')
    names.append('fc_w')
    names.append('fc_b')
    return tuple(names)


_ARG_NAMES = _build_arg_names()


# speedup vs baseline: 1.7440x; 1.0193x over previous
import functools

import jax
import jax.numpy as jnp
from jax import lax
from jax.experimental import pallas as pl
from jax.experimental.pallas import tpu as pltpu

ACT = jnp.bfloat16
EPS = 1e-5
BN_TM = 512


def _ru(x, m):
    return (x + m - 1) // m * m


def _pick_m_tile(M):
    m8 = _ru(M, 8)
    if m8 <= 256:
        return m8, m8
    if M >= 50000 and M % 2048 == 0:
        return 2048, M
    if M >= 50000 and M % 1024 == 0:
        return 1024, M
    return 256, _ru(M, 256)


def _pick_n_tile(N):
    dp = _ru(N, 128)
    for t in (256, 128):
        if dp % t == 0:
            return t, dp
    return 128, dp


def _pick_k_tile(K):
    Kp = _ru(K, 128)
    if Kp <= 640:
        return Kp, Kp
    for t in (512, 384, 256):
        if Kp % t == 0:
            return t, Kp
    return 128, Kp


def _gmm_kernel(a_ref, b_ref, o_ref, acc_ref):
    @pl.when(pl.program_id(3) == 0)
    def _():
        acc_ref[...] = jnp.zeros_like(acc_ref)

    acc_ref[...] += jnp.dot(a_ref[0], b_ref[0],
                            preferred_element_type=jnp.float32)

    @pl.when(pl.program_id(3) == pl.num_programs(3) - 1)
    def _():
        o_ref[0] = acc_ref[...].astype(o_ref.dtype)


def _gmm_stats_kernel(a_ref, b_ref, o_ref, st_ref, acc_ref):
    @pl.when(pl.program_id(3) == 0)
    def _():
        acc_ref[...] = jnp.zeros_like(acc_ref)

    acc_ref[...] += jnp.dot(a_ref[0], b_ref[0],
                            preferred_element_type=jnp.float32)

    @pl.when(pl.program_id(3) == pl.num_programs(3) - 1)
    def _():
        acc = acc_ref[...]
        o_ref[0] = acc.astype(o_ref.dtype)
        colsum = jnp.sum(acc, axis=0, keepdims=True)
        colsq = jnp.sum(acc * acc, axis=0, keepdims=True)
        rows = lax.broadcasted_iota(jnp.int32, st_ref.shape[1:], 0)
        st_ref[0] = jnp.where(rows == 0, colsum,
                              jnp.where(rows == 1, colsq, 0.0))


def gmm(a, b, *, out_dtype=ACT, with_stats=False, shared_a=False):
    Ga, M, K = a.shape
    G, Kb, N = b.shape
    assert K == Kb
    a = a.astype(jnp.bfloat16)
    b = b.astype(jnp.bfloat16)
    tm, Mp = _pick_m_tile(M)
    tn, Np = _pick_n_tile(N)
    tk, Kp = _pick_k_tile(K)
    if Mp != M or Kp != K:
        a = jnp.pad(a, ((0, 0), (0, Mp - M), (0, Kp - K)))
    if Kp != K or Np != N:
        b = jnp.pad(b, ((0, 0), (0, Kp - K), (0, Np - N)))
    grid = (G, Mp // tm, Np // tn, Kp // tk)
    div = G // Ga
    if shared_a:
        a_map = lambda g, i, j, k: (g // div, i, k)
    else:
        a_map = lambda g, i, j, k: (g, i, k)
    cparams = pltpu.CompilerParams(
        dimension_semantics=("parallel", "parallel", "parallel", "arbitrary"))
    in_specs = [pl.BlockSpec((1, tm, tk), a_map),
                pl.BlockSpec((1, tk, tn), lambda g, i, j, k: (g, k, j))]

    if not with_stats:
        out = pl.pallas_call(
            _gmm_kernel,
            out_shape=jax.ShapeDtypeStruct((G, Mp, Np), out_dtype),
            grid_spec=pltpu.PrefetchScalarGridSpec(
                num_scalar_prefetch=0, grid=grid,
                in_specs=in_specs,
                out_specs=pl.BlockSpec((1, tm, tn), lambda g, i, j, k: (g, i, j)),
                scratch_shapes=[pltpu.VMEM((tm, tn), jnp.float32)]),
            compiler_params=cparams,
        )(a, b)
        return out[:, :M, :N]

    mt = Mp // tm
    out, st = pl.pallas_call(
        _gmm_stats_kernel,
        out_shape=(jax.ShapeDtypeStruct((G, Mp, Np), out_dtype),
                   jax.ShapeDtypeStruct((G, mt * 8, Np), jnp.float32)),
        grid_spec=pltpu.PrefetchScalarGridSpec(
            num_scalar_prefetch=0, grid=grid,
            in_specs=in_specs,
            out_specs=(pl.BlockSpec((1, tm, tn), lambda g, i, j, k: (g, i, j)),
                       pl.BlockSpec((1, 8, tn), lambda g, i, j, k: (g, i, j))),
            scratch_shapes=[pltpu.VMEM((tm, tn), jnp.float32)]),
        compiler_params=cparams,
    )(a, b)
    st = st.reshape(G, mt, 8, Np)
    col_sum = jnp.sum(st[:, :, 0, :N], axis=1)
    col_sq = jnp.sum(st[:, :, 1, :N], axis=1)
    mean = col_sum / M
    var = jnp.maximum(col_sq / M - mean * mean, 0.0)
    return out[:, :M, :N], mean, var


def _make_bn_kernel(relu, has_res):
    if has_res:
        def _bn(x_ref, s_ref, b_ref, r_ref, o_ref):
            y = x_ref[0].astype(jnp.float32) * s_ref[0] + b_ref[0]
            y = y + r_ref[0].astype(jnp.float32)
            if relu:
                y = jnp.maximum(y, 0.0)
            o_ref[0] = y.astype(o_ref.dtype)
    else:
        def _bn(x_ref, s_ref, b_ref, o_ref):
            y = x_ref[0].astype(jnp.float32) * s_ref[0] + b_ref[0]
            if relu:
                y = jnp.maximum(y, 0.0)
            o_ref[0] = y.astype(o_ref.dtype)
    return _bn


def gbn(x, mean, var, relu, residual=None):
    G, M, C = x.shape
    scale = (1.0 / jnp.sqrt(var + EPS)).astype(jnp.float32)
    bias = (-mean * scale).astype(jnp.float32)

    fold = 128 // C if (C < 128 and 128 % C == 0) else 1
    Mf = _ru(M, fold)
    Cf = C * fold

    def prep(t):
        if Mf != M:
            t = jnp.pad(t, ((0, 0), (0, Mf - M), (0, 0)))
        return t.reshape(G, Mf // fold, Cf)

    xf = prep(x)
    rf = prep(residual) if residual is not None else None
    rows = Mf // fold
    tm = min(BN_TM, _ru(rows, 8))
    rows_p = _ru(rows, tm)
    if rows_p != rows:
        xf = jnp.pad(xf, ((0, 0), (0, rows_p - rows), (0, 0)))
        if rf is not None:
            rf = jnp.pad(rf, ((0, 0), (0, rows_p - rows), (0, 0)))
    s = jnp.tile(scale.reshape(G, 1, C), (1, 1, fold))
    b = jnp.tile(bias.reshape(G, 1, C), (1, 1, fold))

    in_specs = [pl.BlockSpec((1, tm, Cf), lambda g, i: (g, i, 0)),
                pl.BlockSpec((1, 1, Cf), lambda g, i: (g, 0, 0)),
                pl.BlockSpec((1, 1, Cf), lambda g, i: (g, 0, 0))]
    args = [xf, s, b]
    if rf is not None:
        in_specs.append(pl.BlockSpec((1, tm, Cf), lambda g, i: (g, i, 0)))
        args.append(rf)

    y = pl.pallas_call(
        _make_bn_kernel(relu, rf is not None),
        out_shape=jax.ShapeDtypeStruct((G, rows_p, Cf), ACT),
        grid=(G, rows_p // tm),
        in_specs=in_specs,
        out_specs=pl.BlockSpec((1, tm, Cf), lambda g, i: (g, i, 0)),
        compiler_params=pltpu.CompilerParams(
            dimension_semantics=("parallel", "parallel")),
    )(*args)
    return y[:, :rows].reshape(G, rows * fold, C)[:, :M]


def _make_iconv_kernel(H, W, C):
    def _iconv(x_ref, s_ref, b_ref, w_ref, o_ref, st_ref):
        Bc = x_ref.shape[1]
        xb = x_ref[0].astype(jnp.float32)
        ih = lax.broadcasted_iota(jnp.int32, xb.shape, 1)
        iw = lax.broadcasted_iota(jnp.int32, xb.shape, 2)
        valid = (ih >= 1) & (ih <= H) & (iw >= 1) & (iw <= W)
        xn = jnp.where(valid,
                       jnp.maximum(xb * s_ref[0, 0] + b_ref[0, 0], 0.0),
                       0.0).astype(jnp.bfloat16)
        acc = None
        for t in range(9):
            dy, dx = t // 3, t % 3
            xs = xn[:, dy:dy + H, dx:dx + W, :].reshape(Bc * H * W, C)
            wv = w_ref[0, t * C:(t + 1) * C, :]
            d = jnp.dot(xs, wv, preferred_element_type=jnp.float32)
            acc = d if acc is None else acc + d
        o_ref[0] = acc[:, :o_ref.shape[2]].astype(o_ref.dtype)
        colsum = jnp.sum(acc, axis=0, keepdims=True)
        colsq = jnp.sum(acc * acc, axis=0, keepdims=True)
        rows = lax.broadcasted_iota(jnp.int32, st_ref.shape[1:], 0)
        st_ref[0] = jnp.where(rows == 0, colsum,
                              jnp.where(rows == 1, colsq, 0.0))
    return _iconv


def iconv_g(x, wmat, mean=None, var=None):
    G, B, H, W, C = x.shape
    _, K, N = wmat.shape
    Np = _ru(N, 128)
    w = wmat.astype(jnp.bfloat16)
    if Np != N:
        w = jnp.pad(w, ((0, 0), (0, 0), (0, Np - N)))
    if mean is None:
        scale = jnp.ones((G, 1, C), jnp.float32)
        bias = jnp.zeros((G, 1, C), jnp.float32)
    else:
        s = 1.0 / jnp.sqrt(var + EPS)
        scale = s.reshape(G, 1, C).astype(jnp.float32)
        bias = (-mean * s).reshape(G, 1, C).astype(jnp.float32)
    NC = 1
    while B % (NC * 2) == 0 and (B // NC) * H * W * Np * 4 > 8 * 1024 * 1024:
        NC *= 2
    Bc = B // NC
    xp = jnp.pad(x.astype(jnp.bfloat16),
                 ((0, 0), (0, 0), (1, 1), (1, 1), (0, 0)))
    xp = xp.reshape(G * NC, Bc, H + 2, W + 2, C)
    Mc = Bc * H * W
    out, st = pl.pallas_call(
        _make_iconv_kernel(H, W, C),
        out_shape=(jax.ShapeDtypeStruct((G * NC, Mc, Np), ACT),
                   jax.ShapeDtypeStruct((G * NC, 8, Np), jnp.float32)),
        grid=(G * NC,),
        in_specs=[pl.BlockSpec((1, Bc, H + 2, W + 2, C),
                               lambda i: (i, 0, 0, 0, 0)),
                  pl.BlockSpec((1, 1, C), lambda i: (i // NC, 0, 0)),
                  pl.BlockSpec((1, 1, C), lambda i: (i // NC, 0, 0)),
                  pl.BlockSpec((1, K, Np), lambda i: (i // NC, 0, 0))],
        out_specs=(pl.BlockSpec((1, Mc, Np), lambda i: (i, 0, 0)),
                   pl.BlockSpec((1, 8, Np), lambda i: (i, 0, 0))),
        compiler_params=pltpu.CompilerParams(
            dimension_semantics=("parallel",)),
    )(xp, scale, bias, w)
    M = B * H * W
    out2d = out[:, :, :N].reshape(G, M, N)
    st = st.reshape(G, NC, 8, Np).sum(axis=1)
    mean_o = st[:, 0, :N] / M
    var_o = jnp.maximum(st[:, 1, :N] / M - mean_o * mean_o, 0.0)
    return out2d, mean_o, var_o, (B, H, W, N)


def _make_block_tail_kernel(H, W, C, M):
    HW = H * W

    def _tail(x_ref, s1_ref, b1_ref, w_ref, id_ref, sd_ref, bd_ref, o_ref,
              r2_ref):
        B = x_ref.shape[1]
        s1 = s1_ref[0, 0]
        b1 = b1_ref[0, 0]
        colsum = jnp.zeros((1, C), jnp.float32)
        colsq = jnp.zeros((1, C), jnp.float32)
        for b in range(B):
            xb = x_ref[0, b].astype(jnp.float32)
            ih = lax.broadcasted_iota(jnp.int32, xb.shape, 0)
            iw = lax.broadcasted_iota(jnp.int32, xb.shape, 1)
            valid = (ih >= 1) & (ih <= H) & (iw >= 1) & (iw <= W)
            xn = jnp.where(valid, jnp.maximum(xb * s1 + b1, 0.0),
                           0.0).astype(jnp.bfloat16)
            acc = None
            for t in range(9):
                dy, dx = t // 3, t % 3
                xs = xn[dy:dy + H, dx:dx + W, :].reshape(HW, C)
                wv = w_ref[0, t * C:(t + 1) * C, :]
                d = jnp.dot(xs, wv, preferred_element_type=jnp.float32)
                acc = d if acc is None else acc + d
            acc = acc[:, :C]
            colsum += jnp.sum(acc, axis=0, keepdims=True)
            colsq += jnp.sum(acc * acc, axis=0, keepdims=True)
            r2_ref[b * HW:(b + 1) * HW, :] = acc.astype(jnp.bfloat16)
        mean = colsum / M
        var = jnp.maximum(colsq / M - mean * mean, 0.0)
        s2 = lax.rsqrt(var + EPS)
        b2 = -mean * s2
        for b in range(B):
            rows = slice(b * HW, (b + 1) * HW)
            idv = (id_ref[0, rows, :].astype(jnp.float32) * sd_ref[0, 0]
                   + bd_ref[0, 0])
            y = jnp.maximum(r2_ref[rows, :].astype(jnp.float32) * s2 + b2
                            + idv, 0.0)
            o_ref[0, rows, :] = y.astype(o_ref.dtype)
    return _tail


def block_tail_g(raw1_sp, m1, v1, w2, id2d, id_mean=None, id_var=None):
    G, B, H, W, C = raw1_sp.shape
    _, K, N = w2.shape
    assert N == C
    Np = _ru(N, 128)
    M = B * H * W
    w = w2.astype(jnp.bfloat16)
    if Np != N:
        w = jnp.pad(w, ((0, 0), (0, 0), (0, Np - N)))
    s1 = 1.0 / jnp.sqrt(v1 + EPS)
    scale1 = s1.reshape(G, 1, C).astype(jnp.float32)
    bias1 = (-m1 * s1).reshape(G, 1, C).astype(jnp.float32)
    if id_mean is None:
        sd = jnp.ones((G, 1, C), jnp.float32)
        bd = jnp.zeros((G, 1, C), jnp.float32)
    else:
        sdv = 1.0 / jnp.sqrt(id_var + EPS)
        sd = sdv.reshape(G, 1, C).astype(jnp.float32)
        bd = (-id_mean * sdv).reshape(G, 1, C).astype(jnp.float32)
    xp = jnp.pad(raw1_sp.astype(jnp.bfloat16),
                 ((0, 0), (0, 0), (1, 1), (1, 1), (0, 0)))
    y = pl.pallas_call(
        _make_block_tail_kernel(H, W, C, M),
        out_shape=jax.ShapeDtypeStruct((G, M, C), ACT),
        grid=(G,),
        in_specs=[pl.BlockSpec((1, B, H + 2, W + 2, C),
                               lambda i: (i, 0, 0, 0, 0)),
                  pl.BlockSpec((1, 1, C), lambda i: (i, 0, 0)),
                  pl.BlockSpec((1, 1, C), lambda i: (i, 0, 0)),
                  pl.BlockSpec((1, K, Np), lambda i: (i, 0, 0)),
                  pl.BlockSpec((1, M, C), lambda i: (i, 0, 0)),
                  pl.BlockSpec((1, 1, C), lambda i: (i, 0, 0)),
                  pl.BlockSpec((1, 1, C), lambda i: (i, 0, 0))],
        out_specs=pl.BlockSpec((1, M, C), lambda i: (i, 0, 0)),
        scratch_shapes=[pltpu.VMEM((M, C), jnp.bfloat16)],
        compiler_params=pltpu.CompilerParams(
            dimension_semantics=("parallel",)),
    )(xp, scale1, bias1, w, id2d.astype(jnp.bfloat16), sd, bd)
    return y


def _im2col(x, k, stride, pad, pad_k_to=0):
    G, B, H, W, C = x.shape
    if pad:
        x = jnp.pad(x, ((0, 0), (0, 0), (pad, pad), (pad, pad), (0, 0)))
    Ho = (H + 2 * pad - k) // stride + 1
    Wo = (W + 2 * pad - k) // stride + 1
    cols = []
    for i in range(k):
        for j in range(k):
            cols.append(x[:, :, i:i + stride * Ho:stride,
                          j:j + stride * Wo:stride, :])
    K = k * k * C
    if pad_k_to > K:
        cols.append(jnp.zeros((G, B, Ho, Wo, pad_k_to - K), x.dtype))
        K = pad_k_to
    patches = jnp.concatenate(cols, axis=-1).reshape(G, B * Ho * Wo, K)
    return patches, (B, Ho, Wo)


def conv_g(x, wmat, stride, pad):
    G, B, H, W, C = x.shape
    _, K, Cout = wmat.shape
    k = int(round((K // C) ** 0.5))
    patches, (B_, Ho, Wo) = _im2col(x, k, stride, pad)
    out2d, mean, var = gmm(patches, wmat, with_stats=True)
    return out2d, mean, var, (B_, Ho, Wo, Cout)


def maxpool_3x3_s2(x):
    return lax.reduce_window(x, jnp.asarray(-jnp.inf, x.dtype), lax.max,
                             (1, 1, 3, 3, 1), (1, 1, 2, 2, 1),
                             ((0, 0), (0, 0), (1, 1), (1, 1), (0, 0)))


def basic_block_g(x, wd, w1, w2, stride):
    G, B, H, W, C = x.shape
    if wd is not None:
        id2d, dm, dv, _ = conv_g(x, wd, stride, 0)
    else:
        id2d, dm, dv = x.reshape(G, B * H * W, C), None, None
    if stride == 1:
        raw1, m1, v1, shp = iconv_g(x, w1)
    else:
        raw1, m1, v1, shp = conv_g(x, w1, stride, 1)
    if shp[-1] >= 128:
        y = block_tail_g(raw1.reshape(G, *shp), m1, v1, w2, id2d,
                         id_mean=dm, id_var=dv)
    else:
        if dm is not None:
            id2d = gbn(id2d, dm, dv, relu=False)
        out2d, m2, v2, shp = iconv_g(raw1.reshape(G, *shp), w2,
                                     mean=m1, var=v1)
        y = gbn(out2d, m2, v2, relu=True, residual=id2d)
    return y.reshape(G, *shp)


@jax.jit
def _forward(x1, x2, stacks, fc1_w, fc1_b, fc2_w, fc2_b, fcf_w, fcf_b):
    x1h = jnp.transpose(x1, (0, 2, 3, 1)).astype(ACT)
    x2h = jnp.transpose(x2, (0, 2, 3, 1)).astype(ACT)

    xin = jnp.stack([x1h, x2h])
    patches, (B, Ho, Wo) = _im2col(xin, 7, 2, 3)
    out2d, mean, var = gmm(patches, stacks['conv1_w'], with_stats=True,
                           shared_a=True)
    Cout = stacks['conv1_w'].shape[-1]
    raw = out2d.reshape(4, B, Ho, Wo, Cout)
    pooled = maxpool_3x3_s2(raw)
    G_, B_, Hp_, Wp_, _ = pooled.shape
    x = gbn(pooled.reshape(4, B_ * Hp_ * Wp_, Cout), mean, var,
            relu=True).reshape(4, B_, Hp_, Wp_, Cout)

    for li, (nb, stride) in enumerate(zip((2, 2, 2, 3), (1, 2, 2, 2))):
        for bi in range(nb):
            key = f'layer{li + 1}_{bi}'
            x = basic_block_g(x, stacks.get(key + '_down'),
                              stacks[key + '_conv1'], stacks[key + '_conv2'],
                              stride if bi == 0 else 1)

    pooled = x.astype(jnp.float32).mean(axis=(2, 3))
    a_big = jnp.concatenate([pooled[0], pooled[2], pooled[1], pooled[3]],
                            axis=1)
    n1 = fc1_w.shape[1]
    w_big = jnp.zeros((2048, 3 * n1), jnp.bfloat16)
    w_big = w_big.at[0:512, 0:n1].set(fc1_w.astype(jnp.bfloat16))
    w_big = w_big.at[512:1024, n1:2 * n1].set(fc2_w.astype(jnp.bfloat16))
    w_big = w_big.at[1024:2048, 2 * n1:3 * n1].set(fcf_w.astype(jnp.bfloat16))
    out = gmm(a_big[None], w_big[None], out_dtype=jnp.float32)[0]
    x1_lin = out[:, 0:n1] + fc1_b
    x2_lin = out[:, n1:2 * n1] + fc2_b
    x_out = out[:, 2 * n1:3 * n1] + fcf_b
    return x1_lin, x2_lin, x_out


def kernel(x1, x2, *args):
    names = _ARG_NAMES
    p = dict(zip(names, args))

    def stack4(fmt):
        return jnp.stack([p[fmt.format(net)].astype(jnp.bfloat16)
                          for net in _NETS])

    stacks = {'conv1_w': stack4('{}__conv1_w')}
    for li, nb in enumerate((2, 2, 2, 3)):
        for bi in range(nb):
            key = f'layer{li + 1}_{bi}'
            base = '{}__layer%d__%d__' % (li + 1, bi)
            if bi == 0 and li > 0:
                stacks[key + '_down'] = stack4(base + 'down_w')
            stacks[key + '_conv1'] = stack4(base + 'conv1_w')
            stacks[key + '_conv2'] = stack4(base + 'conv2_w')

    return _forward(x1, x2, stacks,
                    p['ska1__res1__fc_w'], p['ska1__res1__fc_b'],
                    p['ska2__res1__fc_w'], p['ska2__res1__fc_b'],
                    p['fc_w'], p['fc_b'])


_NETS = ('ska1__res1', 'ska1__res2', 'ska2__res1', 'ska2__res2')


def _build_arg_names():
    names = []
    for ska in ('ska1', 'ska2'):
        for res in ('res1', 'res2'):
            pre = f'{ska}__{res}'
            names.append(f'{pre}__conv1_w')
            for li, nb in enumerate((2, 2, 2, 3)):
                for bi in range(nb):
                    if bi == 0 and li > 0:
                        names.append(f'{pre}__layer{li + 1}__{bi}__down_w')
                    names.append(f'{pre}__layer{li + 1}__{bi}__conv1_w')
                    names.append(f'{pre}__layer{li + 1}__{bi}__conv2_w')
            names.append(f'{pre}__fc_w')
            names.append(f'{pre}__fc_b')
    names.append('fc_w')
    names.append('fc_b')
    return tuple(names)


_ARG_NAMES = _build_arg_names()
